# baseline (speedup 1.0000x reference)
import functools

import jax
import jax.numpy as jnp
from jax import lax
from jax.experimental import pallas as pl
from jax.experimental.pallas import tpu as pltpu
from jax.experimental.pallas import tpu_sc as plsc

N_NODES = 10000
N_EDGES = 320000
N_STEPS = 64
EQ_STEPS = 16

NSUB = 16
NCORES = 2
LANES = 16


def _tanh(x):
    e = jnp.exp(2.0 * x)
    return 1.0 - 2.0 / (e + 1.0)


def _make_kernel(n_pad, e_per_w, n_steps, eq_steps):
    seg = n_pad // NSUB
    n_chunks = seg // LANES
    e_chunks = e_per_w // LANES
    total_steps = n_steps + eq_steps

    mesh = plsc.VectorSubcoreMesh(
        core_axis_name="c", subcore_axis_name="s",
        num_cores=NCORES, num_subcores=NSUB)

    @functools.partial(
        pl.kernel,
        out_type=jax.ShapeDtypeStruct((n_steps, n_pad), jnp.float32),
        mesh=mesh,
        compiler_params=pltpu.CompilerParams(needs_layout_passes=False),
        scratch_types=[
            pltpu.VMEM((n_pad,), jnp.float32),
            pltpu.VMEM((e_per_w,), jnp.int32),
            pltpu.VMEM((e_per_w,), jnp.int32),
            pltpu.VMEM((e_per_w,), jnp.float32),
            pltpu.VMEM((n_pad,), jnp.float32),
            pltpu.VMEM((NSUB, seg), jnp.float32),
            pltpu.VMEM((seg,), jnp.float32),
            pltpu.VMEM_SHARED((NSUB, n_pad), jnp.float32),
            pltpu.VMEM_SHARED((n_pad,), jnp.float32),
        ],
    )
    def k(a0_hbm, src_hbm, dst_hbm, w_hbm, out_hbm,
          a_local, src_v, dst_v, w_v, acc, red, act, accs_sh, act_sh):
        cid = lax.axis_index("c")
        sid = lax.axis_index("s")
        base = sid * seg
        ebase = sid * e_per_w

        pltpu.sync_copy(a0_hbm, a_local)
        pltpu.sync_copy(src_hbm.at[pl.ds(ebase, e_per_w)], src_v)
        pltpu.sync_copy(dst_hbm.at[pl.ds(ebase, e_per_w)], dst_v)
        pltpu.sync_copy(w_hbm.at[pl.ds(ebase, e_per_w)], w_v)

        zero16f = jnp.zeros((LANES,), jnp.float32)

        def step(t, _unused):
            def zero_body(i, _):
                acc[pl.ds(i * LANES * 8, LANES)] = zero16f
                acc[pl.ds(i * LANES * 8 + 16, LANES)] = zero16f
                acc[pl.ds(i * LANES * 8 + 32, LANES)] = zero16f
                acc[pl.ds(i * LANES * 8 + 48, LANES)] = zero16f
                acc[pl.ds(i * LANES * 8 + 64, LANES)] = zero16f
                acc[pl.ds(i * LANES * 8 + 80, LANES)] = zero16f
                acc[pl.ds(i * LANES * 8 + 96, LANES)] = zero16f
                acc[pl.ds(i * LANES * 8 + 112, LANES)] = zero16f
                return 0
            lax.fori_loop(0, n_pad // (LANES * 8), zero_body, 0)

            unroll = 8

            def edge_body(cU, _):
                for u in range(unroll):
                    o = (cU * unroll + u) * LANES
                    idx = src_v[pl.ds(o, LANES)]
                    d = dst_v[pl.ds(o, LANES)]
                    wv = w_v[pl.ds(o, LANES)]
                    vals = plsc.load_gather(a_local, [idx])
                    plsc.addupdate_scatter(acc, [d], wv * vals)
                return 0
            lax.fori_loop(0, e_chunks // unroll, edge_body, 0)

            pltpu.sync_copy(acc, accs_sh.at[sid])
            plsc.subcore_barrier()
            for r in range(NSUB):
                pltpu.sync_copy(accs_sh.at[r, pl.ds(base, seg)], red.at[r])

            def act_body(nc, _):
                o = nc * LANES
                agg = red[0, pl.ds(o, LANES)]
                for r in range(1, NSUB):
                    agg = agg + red[r, pl.ds(o, LANES)]
                prev = a_local[pl.ds(base + o, LANES)]
                act[pl.ds(o, LANES)] = _tanh(agg + 0.9 * prev)
                return 0
            lax.fori_loop(0, n_chunks, act_body, 0)

            @pl.when(jnp.logical_and(t >= eq_steps, cid == 0))
            def _():
                pltpu.sync_copy(act, out_hbm.at[t - eq_steps, pl.ds(base, seg)])

            pltpu.sync_copy(act, act_sh.at[pl.ds(base, seg)])
            plsc.subcore_barrier()
            pltpu.sync_copy(act_sh, a_local)
            plsc.subcore_barrier()
            return 0

        lax.fori_loop(0, total_steps, step, 0)

    return k


def kernel(x, edge_index, W):
    n_pad = 10240
    e_per_w = 20480
    e_tot = e_per_w * NSUB
    a0 = jnp.concatenate(
        [x[:, -1], jnp.zeros((n_pad - N_NODES,), jnp.float32)])
    n_real = N_EDGES // NSUB
    pad = e_per_w - n_real
    src = edge_index[0].astype(jnp.int32).reshape(NSUB, n_real)
    dst = edge_index[1].astype(jnp.int32).reshape(NSUB, n_real)
    w2 = W.reshape(NSUB, n_real)
    zi = jnp.zeros((NSUB, pad), jnp.int32)
    zf = jnp.zeros((NSUB, pad), jnp.float32)
    src_g = jnp.concatenate([src, zi], axis=1).reshape(e_tot)
    dst_g = jnp.concatenate([dst, zi], axis=1).reshape(e_tot)
    w_g = jnp.concatenate([w2, zf], axis=1).reshape(e_tot)
    k = _make_kernel(n_pad, e_per_w, N_STEPS, EQ_STEPS)
    out = k(a0, src_g, dst_g, w_g)
    return out[:, :N_NODES]


# speedup vs baseline: 1.0925x; 1.0925x over previous
import functools

import jax
import jax.numpy as jnp
from jax import lax
from jax.experimental import pallas as pl
from jax.experimental.pallas import tpu as pltpu
from jax.experimental.pallas import tpu_sc as plsc

N_NODES = 10000
N_EDGES = 320000
N_STEPS = 64
EQ_STEPS = 16

NSUB = 16
NCORES = 2
LANES = 16


def _tanh(x):
    e = jnp.exp(2.0 * x)
    return 1.0 - 2.0 / (e + 1.0)


def _make_kernel(n_pad, e_per_w, n_steps, eq_steps):
    seg = n_pad // NSUB
    n_chunks = seg // LANES
    e_chunks = e_per_w // LANES
    total_steps = n_steps + eq_steps

    mesh = plsc.VectorSubcoreMesh(
        core_axis_name="c", subcore_axis_name="s",
        num_cores=NCORES, num_subcores=NSUB)

    @functools.partial(
        pl.kernel,
        out_type=jax.ShapeDtypeStruct((n_steps, n_pad), jnp.float32),
        mesh=mesh,
        compiler_params=pltpu.CompilerParams(needs_layout_passes=False),
        scratch_types=[
            pltpu.VMEM((n_pad,), jnp.float32),
            pltpu.VMEM((e_per_w,), jnp.int32),
            pltpu.VMEM((e_per_w,), jnp.int32),
            pltpu.VMEM((e_per_w,), jnp.float32),
            pltpu.VMEM((n_pad,), jnp.float32),
            pltpu.VMEM((NSUB, seg), jnp.float32),
            pltpu.VMEM((seg,), jnp.float32),
            pltpu.VMEM_SHARED((NSUB, n_pad), jnp.float32),
            pltpu.VMEM_SHARED((n_pad,), jnp.float32),
        ],
    )
    def k(a0_hbm, src_hbm, dst_hbm, w_hbm, out_hbm,
          a_local, src_v, dst_v, w_v, acc, red, act, accs_sh, act_sh):
        cid = lax.axis_index("c")
        sid = lax.axis_index("s")
        base = sid * seg
        ebase = sid * e_per_w

        pltpu.sync_copy(a0_hbm, a_local)
        pltpu.sync_copy(src_hbm.at[pl.ds(ebase, e_per_w)], src_v)
        pltpu.sync_copy(dst_hbm.at[pl.ds(ebase, e_per_w)], dst_v)
        pltpu.sync_copy(w_hbm.at[pl.ds(ebase, e_per_w)], w_v)

        zero16f = jnp.zeros((LANES,), jnp.float32)

        def step(t, _unused):
            def zero_body(i, _):
                acc[pl.ds(i * LANES * 8, LANES)] = zero16f
                acc[pl.ds(i * LANES * 8 + 16, LANES)] = zero16f
                acc[pl.ds(i * LANES * 8 + 32, LANES)] = zero16f
                acc[pl.ds(i * LANES * 8 + 48, LANES)] = zero16f
                acc[pl.ds(i * LANES * 8 + 64, LANES)] = zero16f
                acc[pl.ds(i * LANES * 8 + 80, LANES)] = zero16f
                acc[pl.ds(i * LANES * 8 + 96, LANES)] = zero16f
                acc[pl.ds(i * LANES * 8 + 112, LANES)] = zero16f
                return 0
            lax.fori_loop(0, n_pad // (LANES * 8), zero_body, 0)

            unroll = 8

            def edge_body(cU, _):
                for u in range(unroll):
                    o = (cU * unroll + u) * LANES
                    idx = src_v[pl.ds(o, LANES)]
                    d = dst_v[pl.ds(o, LANES)]
                    wv = w_v[pl.ds(o, LANES)]
                    vals = plsc.load_gather(a_local, [idx])
                    plsc.addupdate_scatter(acc, [d], wv * vals)
                return 0
            lax.fori_loop(0, e_chunks // unroll, edge_body, 0)

            pltpu.sync_copy(acc, accs_sh.at[sid])
            plsc.subcore_barrier()
            pltpu.sync_copy(accs_sh.at[:, pl.ds(base, seg)], red)

            def act_body(nc, _):
                o = nc * LANES
                vs = [red[r, pl.ds(o, LANES)] for r in range(NSUB)]
                while len(vs) > 1:
                    vs = [a + b for a, b in zip(vs[::2], vs[1::2])]
                agg = vs[0]
                prev = a_local[pl.ds(base + o, LANES)]
                act[pl.ds(o, LANES)] = _tanh(agg + 0.9 * prev)
                return 0
            lax.fori_loop(0, n_chunks, act_body, 0)

            @pl.when(jnp.logical_and(t >= eq_steps, cid == 0))
            def _():
                pltpu.sync_copy(act, out_hbm.at[t - eq_steps, pl.ds(base, seg)])

            pltpu.sync_copy(act, act_sh.at[pl.ds(base, seg)])
            plsc.subcore_barrier()
            pltpu.sync_copy(act_sh, a_local)
            plsc.subcore_barrier()
            return 0

        lax.fori_loop(0, total_steps, step, 0)

    return k


def kernel(x, edge_index, W):
    n_pad = 10240
    e_per_w = 20480
    e_tot = e_per_w * NSUB
    a0 = jnp.concatenate(
        [x[:, -1], jnp.zeros((n_pad - N_NODES,), jnp.float32)])
    n_real = N_EDGES // NSUB
    pad = e_per_w - n_real
    src = edge_index[0].astype(jnp.int32).reshape(NSUB, n_real)
    dst = edge_index[1].astype(jnp.int32).reshape(NSUB, n_real)
    w2 = W.reshape(NSUB, n_real)
    zi = jnp.zeros((NSUB, pad), jnp.int32)
    zf = jnp.zeros((NSUB, pad), jnp.float32)
    src_g = jnp.concatenate([src, zi], axis=1).reshape(e_tot)
    dst_g = jnp.concatenate([dst, zi], axis=1).reshape(e_tot)
    w_g = jnp.concatenate([w2, zf], axis=1).reshape(e_tot)
    k = _make_kernel(n_pad, e_per_w, N_STEPS, EQ_STEPS)
    out = k(a0, src_g, dst_g, w_g)
    return out[:, :N_NODES]
